# baseline (speedup 1.0000x reference)
import functools

import jax
import jax.numpy as jnp
from jax import lax
from jax.experimental import pallas as pl
from jax.experimental.pallas import tpu as pltpu
from jax.experimental.pallas import tpu_sc as plsc

N_NODES = 10000
N_EDGES = 320000
D_FEAT = 128

NC = 2
NS = 16
NW = NC * NS
CH = 128
CHUNKS_PER_W = 80
E_PAD = NW * CHUNKS_PER_W * CH
N_PAD = 10240
STRIPE = N_PAD // NS
DUMMY = N_NODES


def _unpack(pbuf, base, gbuf, mbuf):
    for i in range(CH // 16):
        v = pbuf[pl.ds(base + i * 16, 16)]
        r = v & jnp.int32(0xFFFF)
        cc = lax.shift_right_logical(v, jnp.int32(16))
        gbuf[pl.ds(i * 16, 16)] = r
        mbuf[pl.ds(i * 16, 16)] = jnp.where(
            r == cc, jnp.full((16,), DUMMY, jnp.int32), cc)


def _deg_body(pk_hbm, out_hbm, pbuf, gbuf, mbuf, vbuf, zbuf, cnt_sh):
    c = lax.axis_index("c")
    s = lax.axis_index("s")
    w = s * NC + c

    def _z(i, carry):
        zbuf[pl.ds(i * 16, 16)] = jnp.zeros((16,), jnp.float32)
        return carry
    lax.fori_loop(jnp.int32(0), jnp.int32(STRIPE // 16), _z, jnp.int32(0))
    pltpu.sync_copy(zbuf, cnt_sh.at[pl.ds(s * STRIPE, STRIPE)])

    for i in range(CH // 16):
        vbuf[pl.ds(i * 16, 16)] = jnp.ones((16,), jnp.float32)

    nloc = CHUNKS_PER_W * CH
    pltpu.sync_copy(pk_hbm.at[pl.ds(w * nloc, nloc)], pbuf)

    plsc.subcore_barrier()

    def _step(k, carry):
        _unpack(pbuf, k * CH, gbuf, mbuf)
        pltpu.sync_copy(vbuf, cnt_sh.at[mbuf], add=True)
        return carry
    lax.fori_loop(jnp.int32(0), jnp.int32(CHUNKS_PER_W), _step, jnp.int32(0))

    plsc.subcore_barrier()
    pltpu.sync_copy(cnt_sh.at[pl.ds(s * STRIPE, STRIPE)],
                    out_hbm.at[c].at[pl.ds(s * STRIPE, STRIPE)])


_deg_kernel = functools.partial(
    pl.kernel,
    out_type=jax.ShapeDtypeStruct((NC, N_PAD), jnp.float32),
    mesh=plsc.VectorSubcoreMesh(core_axis_name="c", subcore_axis_name="s"),
    scratch_types=[
        pltpu.VMEM((CHUNKS_PER_W * CH,), jnp.int32),
        pltpu.VMEM((CH,), jnp.int32),
        pltpu.VMEM((CH,), jnp.int32),
        pltpu.VMEM((CH,), jnp.float32),
        pltpu.VMEM((STRIPE,), jnp.float32),
        pltpu.VMEM_SHARED((N_PAD,), jnp.float32),
    ],
    compiler_params=pltpu.CompilerParams(use_tc_tiling_on_sc=False),
)(_deg_body)


def _prop_body(pk_hbm, y_hbm, out_hbm,
               pbuf, gbufa, gbufb, mbufa, mbufb, rowa, rowb, zbuf, acc_sh,
               sga, sgb, ssa, ssb):
    c = lax.axis_index("c")
    s = lax.axis_index("s")
    w = s * NC + c

    for r in range(16):
        for j in range(D_FEAT // 16):
            zbuf[r, pl.ds(j * 16, 16)] = jnp.zeros((16,), jnp.float32)

    def _z(i, carry):
        pltpu.sync_copy(zbuf, acc_sh.at[pl.ds(s * STRIPE + i * 16, 16)])
        return carry
    lax.fori_loop(jnp.int32(0), jnp.int32(STRIPE // 16), _z, jnp.int32(0))

    nloc = CHUNKS_PER_W * CH
    pltpu.sync_copy(pk_hbm.at[pl.ds(w * nloc, nloc)], pbuf)

    plsc.subcore_barrier()

    def _gather_start(gbuf, buf, sem):
        pltpu.async_copy(y_hbm.at[gbuf], buf, sem)

    def _gather_wait(gbuf, buf, sem):
        pltpu.make_async_copy(y_hbm.at[gbuf], buf, sem).wait()

    def _scatter_start(buf, mbuf, sem):
        pltpu.async_copy(buf, acc_sh.at[mbuf], sem, add=True)

    def _scatter_wait(buf, mbuf, sem):
        pltpu.make_async_copy(buf, acc_sh.at[mbuf], sem).wait()

    _unpack(pbuf, jnp.int32(0), gbufa, mbufa)
    _gather_start(gbufa, rowa, sga)
    _unpack(pbuf, jnp.int32(CH), gbufb, mbufb)
    _gather_start(gbufb, rowb, sgb)

    def _step(j, carry):
        k0 = 2 * j
        _gather_wait(gbufa, rowa, sga)
        _scatter_start(rowa, mbufa, ssa)

        _gather_wait(gbufb, rowb, sgb)
        _scatter_start(rowb, mbufb, ssb)

        @pl.when(j < CHUNKS_PER_W // 2 - 1)
        def _():
            _scatter_wait(rowa, mbufa, ssa)
            _unpack(pbuf, (k0 + 2) * CH, gbufa, mbufa)
            _gather_start(gbufa, rowa, sga)

            _scatter_wait(rowb, mbufb, ssb)
            _unpack(pbuf, (k0 + 3) * CH, gbufb, mbufb)
            _gather_start(gbufb, rowb, sgb)
        return carry
    lax.fori_loop(jnp.int32(0), jnp.int32(CHUNKS_PER_W // 2), _step,
                  jnp.int32(0))

    _scatter_wait(rowa, mbufa, ssa)
    _scatter_wait(rowb, mbufb, ssb)

    plsc.subcore_barrier()

    def _out(i, carry):
        pltpu.sync_copy(acc_sh.at[pl.ds(s * STRIPE + i * CH, CH)], rowa)
        pltpu.sync_copy(rowa, out_hbm.at[c].at[pl.ds(s * STRIPE + i * CH, CH)])
        return carry
    lax.fori_loop(jnp.int32(0), jnp.int32(STRIPE // CH), _out, jnp.int32(0))


_prop_kernel = functools.partial(
    pl.kernel,
    out_type=jax.ShapeDtypeStruct((NC, N_PAD, D_FEAT), jnp.float32),
    mesh=plsc.VectorSubcoreMesh(core_axis_name="c", subcore_axis_name="s"),
    scratch_types=[
        pltpu.VMEM((CHUNKS_PER_W * CH,), jnp.int32),
        pltpu.VMEM((CH,), jnp.int32),
        pltpu.VMEM((CH,), jnp.int32),
        pltpu.VMEM((CH,), jnp.int32),
        pltpu.VMEM((CH,), jnp.int32),
        pltpu.VMEM((CH, D_FEAT), jnp.float32),
        pltpu.VMEM((CH, D_FEAT), jnp.float32),
        pltpu.VMEM((16, D_FEAT), jnp.float32),
        pltpu.VMEM_SHARED((N_PAD, D_FEAT), jnp.float32),
        pltpu.SemaphoreType.DMA,
        pltpu.SemaphoreType.DMA,
        pltpu.SemaphoreType.DMA,
        pltpu.SemaphoreType.DMA,
    ],
    compiler_params=pltpu.CompilerParams(use_tc_tiling_on_sc=False),
)(_prop_body)


def _e1_body(cnt_ref, dinv_ref):
    deg = cnt_ref[0] + cnt_ref[1] + 1.0
    dinv_ref[...] = lax.rsqrt(deg)


def _e2_body(x_ref, dinv_ref, y_ref):
    y_ref[...] = x_ref[...] * dinv_ref[...]


def _e3_body(p_ref, x_ref, dinv_ref, out_ref):
    dinv = dinv_ref[...]
    out_ref[...] = dinv * (p_ref[0] + p_ref[1]) + dinv * dinv * x_ref[...]


def kernel(x, edge_index):
    ei = edge_index.astype(jnp.int32)
    row, col = ei[0], ei[1]
    pad = E_PAD - N_EDGES
    row = jnp.concatenate([row, jnp.zeros((pad,), jnp.int32)])
    col = jnp.concatenate([col, jnp.full((pad,), DUMMY, jnp.int32)])
    packed = jnp.bitwise_or(jnp.left_shift(col, 16), row)

    cnt = _deg_kernel(packed)

    dinv3 = pl.pallas_call(
        _e1_body,
        out_shape=jax.ShapeDtypeStruct((N_PAD // 128, 128), jnp.float32),
    )(cnt.reshape(NC, N_PAD // 128, 128))
    dinv_col = dinv3.reshape(N_PAD)[:N_NODES, None]

    y = pl.pallas_call(
        _e2_body,
        out_shape=jax.ShapeDtypeStruct((N_NODES, D_FEAT), jnp.float32),
    )(x, dinv_col)

    p = _prop_kernel(packed, y)

    out = pl.pallas_call(
        _e3_body,
        out_shape=jax.ShapeDtypeStruct((N_NODES, D_FEAT), jnp.float32),
    )(p[:, :N_NODES, :], x, dinv_col)
    return out


# speedup vs baseline: 1.0440x; 1.0440x over previous
import functools

import jax
import jax.numpy as jnp
from jax import lax
from jax.experimental import pallas as pl
from jax.experimental.pallas import tpu as pltpu
from jax.experimental.pallas import tpu_sc as plsc

N_NODES = 10000
N_EDGES = 320000
D_FEAT = 128

NC = 2
NS = 16
NW = NC * NS
CH = 128
CHUNKS_PER_W = 80
E_PAD = NW * CHUNKS_PER_W * CH
N_PAD = 10240
STRIPE = N_PAD // NS
DUMMY = N_NODES


def _unpack(pbuf, base, gbuf, mbuf):
    for i in range(CH // 16):
        v = pbuf[pl.ds(base + i * 16, 16)]
        r = v & jnp.int32(0xFFFF)
        cc = lax.shift_right_logical(v, jnp.int32(16))
        gbuf[pl.ds(i * 16, 16)] = r
        mbuf[pl.ds(i * 16, 16)] = jnp.where(
            r == cc, jnp.full((16,), DUMMY, jnp.int32), cc)


def _deg_body(pk_hbm, out_hbm, pbuf, gbuf, mbuf, vbuf, zbuf, cnt_sh):
    c = lax.axis_index("c")
    s = lax.axis_index("s")
    w = s * NC + c

    def _z(i, carry):
        zbuf[pl.ds(i * 16, 16)] = jnp.zeros((16,), jnp.float32)
        return carry
    lax.fori_loop(jnp.int32(0), jnp.int32(STRIPE // 16), _z, jnp.int32(0))
    pltpu.sync_copy(zbuf, cnt_sh.at[pl.ds(s * STRIPE, STRIPE)])

    for i in range(CH // 16):
        vbuf[pl.ds(i * 16, 16)] = jnp.ones((16,), jnp.float32)

    nloc = CHUNKS_PER_W * CH
    pltpu.sync_copy(pk_hbm.at[pl.ds(w * nloc, nloc)], pbuf)

    plsc.subcore_barrier()

    def _step(k, carry):
        _unpack(pbuf, k * CH, gbuf, mbuf)
        pltpu.sync_copy(vbuf, cnt_sh.at[mbuf], add=True)
        return carry
    lax.fori_loop(jnp.int32(0), jnp.int32(CHUNKS_PER_W), _step, jnp.int32(0))

    plsc.subcore_barrier()
    pltpu.sync_copy(cnt_sh.at[pl.ds(s * STRIPE, STRIPE)],
                    out_hbm.at[c].at[pl.ds(s * STRIPE, STRIPE)])


_deg_kernel = functools.partial(
    pl.kernel,
    out_type=jax.ShapeDtypeStruct((NC, N_PAD), jnp.float32),
    mesh=plsc.VectorSubcoreMesh(core_axis_name="c", subcore_axis_name="s"),
    scratch_types=[
        pltpu.VMEM((CHUNKS_PER_W * CH,), jnp.int32),
        pltpu.VMEM((CH,), jnp.int32),
        pltpu.VMEM((CH,), jnp.int32),
        pltpu.VMEM((CH,), jnp.float32),
        pltpu.VMEM((STRIPE,), jnp.float32),
        pltpu.VMEM_SHARED((N_PAD,), jnp.float32),
    ],
    compiler_params=pltpu.CompilerParams(use_tc_tiling_on_sc=False),
)(_deg_body)


def _prop_body(pk_hbm, y_hbm, out_hbm,
               pbuf, gbufa, gbufb, mbufa, mbufb, rowa, rowb, zbuf, acc_sh,
               sga, sgb, ssa, ssb):
    c = lax.axis_index("c")
    s = lax.axis_index("s")
    w = s * NC + c

    for r in range(16):
        for j in range(D_FEAT // 16):
            zbuf[r, pl.ds(j * 16, 16)] = jnp.zeros((16,), jnp.float32)

    def _z(i, carry):
        pltpu.sync_copy(zbuf, acc_sh.at[pl.ds(s * STRIPE + i * 16, 16)])
        return carry
    lax.fori_loop(jnp.int32(0), jnp.int32(STRIPE // 16), _z, jnp.int32(0))

    nloc = CHUNKS_PER_W * CH
    pltpu.sync_copy(pk_hbm.at[pl.ds(w * nloc, nloc)], pbuf)

    plsc.subcore_barrier()

    def _gather_start(gbuf, buf, sem):
        pltpu.async_copy(y_hbm.at[gbuf], buf, sem)

    def _gather_wait(gbuf, buf, sem):
        pltpu.make_async_copy(y_hbm.at[gbuf], buf, sem).wait()

    def _scatter_start(buf, mbuf, sem):
        pltpu.async_copy(buf, acc_sh.at[mbuf], sem, add=True)

    def _scatter_wait(buf, mbuf, sem):
        pltpu.make_async_copy(buf, acc_sh.at[mbuf], sem).wait()

    _unpack(pbuf, jnp.int32(0), gbufa, mbufa)
    _gather_start(gbufa, rowa, sga)
    _unpack(pbuf, jnp.int32(CH), gbufb, mbufb)
    _gather_start(gbufb, rowb, sgb)

    def _step(j, carry):
        k0 = 2 * j
        _gather_wait(gbufa, rowa, sga)
        _scatter_start(rowa, mbufa, ssa)

        _gather_wait(gbufb, rowb, sgb)
        _scatter_start(rowb, mbufb, ssb)

        @pl.when(j < CHUNKS_PER_W // 2 - 1)
        def _():
            _scatter_wait(rowa, mbufa, ssa)
            _unpack(pbuf, (k0 + 2) * CH, gbufa, mbufa)
            _gather_start(gbufa, rowa, sga)

            _scatter_wait(rowb, mbufb, ssb)
            _unpack(pbuf, (k0 + 3) * CH, gbufb, mbufb)
            _gather_start(gbufb, rowb, sgb)
        return carry
    lax.fori_loop(jnp.int32(0), jnp.int32(CHUNKS_PER_W // 2), _step,
                  jnp.int32(0))

    _scatter_wait(rowa, mbufa, ssa)
    _scatter_wait(rowb, mbufb, ssb)

    plsc.subcore_barrier()

    def _out(i, carry):
        pltpu.sync_copy(acc_sh.at[pl.ds(s * STRIPE + i * CH, CH)], rowa)
        pltpu.sync_copy(rowa, out_hbm.at[c].at[pl.ds(s * STRIPE + i * CH, CH)])
        return carry
    lax.fori_loop(jnp.int32(0), jnp.int32(STRIPE // CH), _out, jnp.int32(0))


_prop_kernel = functools.partial(
    pl.kernel,
    out_type=jax.ShapeDtypeStruct((NC, N_PAD, D_FEAT), jnp.float32),
    mesh=plsc.VectorSubcoreMesh(core_axis_name="c", subcore_axis_name="s"),
    scratch_types=[
        pltpu.VMEM((CHUNKS_PER_W * CH,), jnp.int32),
        pltpu.VMEM((CH,), jnp.int32),
        pltpu.VMEM((CH,), jnp.int32),
        pltpu.VMEM((CH,), jnp.int32),
        pltpu.VMEM((CH,), jnp.int32),
        pltpu.VMEM((CH, D_FEAT), jnp.float32),
        pltpu.VMEM((CH, D_FEAT), jnp.float32),
        pltpu.VMEM((16, D_FEAT), jnp.float32),
        pltpu.VMEM_SHARED((N_PAD, D_FEAT), jnp.float32),
        pltpu.SemaphoreType.DMA,
        pltpu.SemaphoreType.DMA,
        pltpu.SemaphoreType.DMA,
        pltpu.SemaphoreType.DMA,
    ],
    compiler_params=pltpu.CompilerParams(use_tc_tiling_on_sc=False),
)(_prop_body)


def _e1_body(cnt_ref, dinv_ref):
    deg = cnt_ref[0] + cnt_ref[1] + 1.0
    dinv_ref[...] = lax.rsqrt(deg)


def _e2_body(x_ref, dinv_ref, y_ref):
    y_ref[...] = x_ref[...] * dinv_ref[...]


def _e3_body(p_ref, x_ref, dinv_ref, out_ref):
    dinv = dinv_ref[...]
    out_ref[...] = dinv * (p_ref[0] + p_ref[1]) + dinv * dinv * x_ref[...]


def kernel(x, edge_index):
    ei = edge_index.astype(jnp.int32)
    row, col = ei[0], ei[1]
    pad = E_PAD - N_EDGES
    pad_col = DUMMY + (jnp.arange(pad, dtype=jnp.int32) % (N_PAD - N_NODES))
    row = jnp.concatenate([row, jnp.zeros((pad,), jnp.int32)])
    col = jnp.concatenate([col, pad_col])
    packed = jnp.bitwise_or(jnp.left_shift(col, 16), row)

    cnt = _deg_kernel(packed)

    dinv3 = pl.pallas_call(
        _e1_body,
        out_shape=jax.ShapeDtypeStruct((N_PAD // 128, 128), jnp.float32),
    )(cnt.reshape(NC, N_PAD // 128, 128))
    dinv_col = dinv3.reshape(N_PAD)[:N_NODES, None]

    y = pl.pallas_call(
        _e2_body,
        out_shape=jax.ShapeDtypeStruct((N_NODES, D_FEAT), jnp.float32),
    )(x, dinv_col)

    p = _prop_kernel(packed, y)

    out = pl.pallas_call(
        _e3_body,
        out_shape=jax.ShapeDtypeStruct((N_NODES, D_FEAT), jnp.float32),
    )(p[:, :N_NODES, :], x, dinv_col)
    return out


# speedup vs baseline: 2.3446x; 2.2458x over previous
import functools

import jax
import jax.numpy as jnp
from jax import lax
from jax.experimental import pallas as pl
from jax.experimental.pallas import tpu as pltpu
from jax.experimental.pallas import tpu_sc as plsc

N_NODES = 10000
N_EDGES = 320000
D_FEAT = 128

NC = 2
NS = 16
NW = NC * NS
CH = 128
CHUNKS_PER_W = 80
E_PAD = NW * CHUNKS_PER_W * CH
N_PAD = 10240
STRIPE = N_PAD // NS
DUMMY = N_NODES


def _unpack(pbuf, base, gbuf, mbuf):
    for i in range(CH // 16):
        v = pbuf[pl.ds(base + i * 16, 16)]
        r = v & jnp.int32(0xFFFF)
        cc = lax.shift_right_logical(v, jnp.int32(16))
        gbuf[pl.ds(i * 16, 16)] = r
        mbuf[pl.ds(i * 16, 16)] = jnp.where(
            r == cc, jnp.full((16,), DUMMY, jnp.int32), cc)


def _deg_body(pk_hbm, out_hbm, pbuf, gbuf, mbuf, vbuf, zbuf, cnt_sh):
    c = lax.axis_index("c")
    s = lax.axis_index("s")
    w = s * NC + c

    def _z(i, carry):
        zbuf[pl.ds(i * 16, 16)] = jnp.zeros((16,), jnp.float32)
        return carry
    lax.fori_loop(jnp.int32(0), jnp.int32(STRIPE // 16), _z, jnp.int32(0))
    pltpu.sync_copy(zbuf, cnt_sh.at[pl.ds(s * STRIPE, STRIPE)])

    for i in range(CH // 16):
        vbuf[pl.ds(i * 16, 16)] = jnp.ones((16,), jnp.float32)

    nloc = CHUNKS_PER_W * CH
    pltpu.sync_copy(pk_hbm.at[pl.ds(w * nloc, nloc)], pbuf)

    plsc.subcore_barrier()

    def _step(k, carry):
        _unpack(pbuf, k * CH, gbuf, mbuf)
        pltpu.sync_copy(vbuf, cnt_sh.at[mbuf], add=True)
        return carry
    lax.fori_loop(jnp.int32(0), jnp.int32(CHUNKS_PER_W), _step, jnp.int32(0))

    plsc.subcore_barrier()
    pltpu.sync_copy(cnt_sh.at[pl.ds(s * STRIPE, STRIPE)],
                    out_hbm.at[c].at[pl.ds(s * STRIPE, STRIPE)])


_deg_kernel = functools.partial(
    pl.kernel,
    out_type=jax.ShapeDtypeStruct((NC, N_PAD), jnp.float32),
    mesh=plsc.VectorSubcoreMesh(core_axis_name="c", subcore_axis_name="s"),
    scratch_types=[
        pltpu.VMEM((CHUNKS_PER_W * CH,), jnp.int32),
        pltpu.VMEM((CH,), jnp.int32),
        pltpu.VMEM((CH,), jnp.int32),
        pltpu.VMEM((CH,), jnp.float32),
        pltpu.VMEM((STRIPE,), jnp.float32),
        pltpu.VMEM_SHARED((N_PAD,), jnp.float32),
    ],
    compiler_params=pltpu.CompilerParams(use_tc_tiling_on_sc=False),
)(_deg_body)


DH = D_FEAT // 2
CHUNKS_B = E_PAD // (NS * CH)
NROW_T = N_NODES // NS


def _prop_body(pk_hbm, y_hbm, out_hbm,
               pbuf, gbufa, gbufb, mbufa, mbufb, rowa, rowb, zbuf,
               y_sh, acc_sh, sga, sgb, ssa, ssb):
    c = lax.axis_index("c")
    s = lax.axis_index("s")

    for r in range(16):
        for j in range(DH // 16):
            zbuf[r, pl.ds(j * 16, 16)] = jnp.zeros((16,), jnp.float32)

    def _z(i, carry):
        pltpu.sync_copy(zbuf, acc_sh.at[pl.ds(s * STRIPE + i * 16, 16)])
        return carry
    lax.fori_loop(jnp.int32(0), jnp.int32(STRIPE // 16), _z, jnp.int32(0))

    pltpu.sync_copy(y_hbm.at[c].at[pl.ds(s * NROW_T, NROW_T)],
                    y_sh.at[pl.ds(s * NROW_T, NROW_T)])

    nloc = CHUNKS_B * CH
    pltpu.sync_copy(pk_hbm.at[pl.ds(s * nloc, nloc)], pbuf)

    plsc.subcore_barrier()

    def _gather_start(gbuf, buf, sem):
        pltpu.async_copy(y_sh.at[gbuf], buf, sem)

    def _gather_wait(gbuf, buf, sem):
        pltpu.make_async_copy(y_sh.at[gbuf], buf, sem).wait()

    def _scatter_start(buf, mbuf, sem):
        pltpu.async_copy(buf, acc_sh.at[mbuf], sem, add=True)

    def _scatter_wait(buf, mbuf, sem):
        pltpu.make_async_copy(buf, acc_sh.at[mbuf], sem).wait()

    _unpack(pbuf, jnp.int32(0), gbufa, mbufa)
    _gather_start(gbufa, rowa, sga)
    _unpack(pbuf, jnp.int32(CH), gbufb, mbufb)
    _gather_start(gbufb, rowb, sgb)

    def _step(j, carry):
        k0 = 2 * j
        _gather_wait(gbufa, rowa, sga)
        _scatter_start(rowa, mbufa, ssa)

        _gather_wait(gbufb, rowb, sgb)
        _scatter_start(rowb, mbufb, ssb)

        @pl.when(j < CHUNKS_B // 2 - 1)
        def _():
            _scatter_wait(rowa, mbufa, ssa)
            _unpack(pbuf, (k0 + 2) * CH, gbufa, mbufa)
            _gather_start(gbufa, rowa, sga)

            _scatter_wait(rowb, mbufb, ssb)
            _unpack(pbuf, (k0 + 3) * CH, gbufb, mbufb)
            _gather_start(gbufb, rowb, sgb)
        return carry
    lax.fori_loop(jnp.int32(0), jnp.int32(CHUNKS_B // 2), _step, jnp.int32(0))

    _scatter_wait(rowa, mbufa, ssa)
    _scatter_wait(rowb, mbufb, ssb)

    plsc.subcore_barrier()

    def _out(i, carry):
        pltpu.sync_copy(acc_sh.at[pl.ds(s * STRIPE + i * CH, CH)], rowa)
        pltpu.sync_copy(rowa, out_hbm.at[c].at[pl.ds(s * STRIPE + i * CH, CH)])
        return carry
    lax.fori_loop(jnp.int32(0), jnp.int32(STRIPE // CH), _out, jnp.int32(0))


_prop_kernel = functools.partial(
    pl.kernel,
    out_type=jax.ShapeDtypeStruct((NC, N_PAD, DH), jnp.float32),
    mesh=plsc.VectorSubcoreMesh(core_axis_name="c", subcore_axis_name="s"),
    scratch_types=[
        pltpu.VMEM((CHUNKS_B * CH,), jnp.int32),
        pltpu.VMEM((CH,), jnp.int32),
        pltpu.VMEM((CH,), jnp.int32),
        pltpu.VMEM((CH,), jnp.int32),
        pltpu.VMEM((CH,), jnp.int32),
        pltpu.VMEM((CH, DH), jnp.float32),
        pltpu.VMEM((CH, DH), jnp.float32),
        pltpu.VMEM((16, DH), jnp.float32),
        pltpu.VMEM_SHARED((N_NODES, DH), jnp.float32),
        pltpu.VMEM_SHARED((N_PAD, DH), jnp.float32),
        pltpu.SemaphoreType.DMA,
        pltpu.SemaphoreType.DMA,
        pltpu.SemaphoreType.DMA,
        pltpu.SemaphoreType.DMA,
    ],
    compiler_params=pltpu.CompilerParams(use_tc_tiling_on_sc=False),
)(_prop_body)


def _e1_body(cnt_ref, dinv_ref):
    deg = cnt_ref[0] + cnt_ref[1] + 1.0
    dinv_ref[...] = lax.rsqrt(deg)


def _e2_body(x_ref, dinv_ref, y_ref):
    d = dinv_ref[...]
    y_ref[0] = x_ref[:, 0:64] * d
    y_ref[1] = x_ref[:, 64:128] * d


def _e3_body(p_ref, x_ref, dinv_ref, out_ref):
    dinv = dinv_ref[...]
    x = x_ref[...]
    out_ref[:, 0:64] = dinv * p_ref[0] + dinv * dinv * x[:, 0:64]
    out_ref[:, 64:128] = dinv * p_ref[1] + dinv * dinv * x[:, 64:128]


def kernel(x, edge_index):
    ei = edge_index.astype(jnp.int32)
    row, col = ei[0], ei[1]
    pad = E_PAD - N_EDGES
    pad_col = DUMMY + (jnp.arange(pad, dtype=jnp.int32) % (N_PAD - N_NODES))
    row = jnp.concatenate([row, jnp.zeros((pad,), jnp.int32)])
    col = jnp.concatenate([col, pad_col])
    packed = jnp.bitwise_or(jnp.left_shift(col, 16), row)

    cnt = _deg_kernel(packed)

    dinv3 = pl.pallas_call(
        _e1_body,
        out_shape=jax.ShapeDtypeStruct((N_PAD // 128, 128), jnp.float32),
    )(cnt.reshape(NC, N_PAD // 128, 128))
    dinv_col = dinv3.reshape(N_PAD)[:N_NODES, None]

    y = pl.pallas_call(
        _e2_body,
        out_shape=jax.ShapeDtypeStruct((NC, N_NODES, 64), jnp.float32),
    )(x, dinv_col)

    p = _prop_kernel(packed, y)

    out = pl.pallas_call(
        _e3_body,
        out_shape=jax.ShapeDtypeStruct((N_NODES, D_FEAT), jnp.float32),
    )(p[:, :N_NODES, :], x, dinv_col)
    return out


# speedup vs baseline: 2.4500x; 1.0449x over previous
import functools

import jax
import jax.numpy as jnp
from jax import lax
from jax.experimental import pallas as pl
from jax.experimental.pallas import tpu as pltpu
from jax.experimental.pallas import tpu_sc as plsc

N_NODES = 10000
N_EDGES = 320000
D_FEAT = 128

NC = 2
NS = 16
NW = NC * NS
CH = 128
CHUNKS_PER_W = 80
E_PAD = NW * CHUNKS_PER_W * CH
N_PAD = 10240
STRIPE = N_PAD // NS
DUMMY = N_NODES


def _unpack(pbuf, base, gbuf, mbuf):
    for i in range(CH // 16):
        v = pbuf[pl.ds(base + i * 16, 16)]
        r = v & jnp.int32(0xFFFF)
        cc = lax.shift_right_logical(v, jnp.int32(16))
        gbuf[pl.ds(i * 16, 16)] = r
        mbuf[pl.ds(i * 16, 16)] = jnp.where(
            r == cc, jnp.full((16,), DUMMY, jnp.int32), cc)


def _deg_body(pk_hbm, out_hbm, pbuf, gbuf, mbuf, vbuf, zbuf, cnt_sh):
    c = lax.axis_index("c")
    s = lax.axis_index("s")
    w = s * NC + c

    def _z(i, carry):
        zbuf[pl.ds(i * 16, 16)] = jnp.zeros((16,), jnp.float32)
        return carry
    lax.fori_loop(jnp.int32(0), jnp.int32(STRIPE // 16), _z, jnp.int32(0))
    pltpu.sync_copy(zbuf, cnt_sh.at[pl.ds(s * STRIPE, STRIPE)])

    for i in range(CH // 16):
        vbuf[pl.ds(i * 16, 16)] = jnp.ones((16,), jnp.float32)

    nloc = CHUNKS_PER_W * CH
    pltpu.sync_copy(pk_hbm.at[pl.ds(w * nloc, nloc)], pbuf)

    plsc.subcore_barrier()

    def _step(k, carry):
        _unpack(pbuf, k * CH, gbuf, mbuf)
        pltpu.sync_copy(vbuf, cnt_sh.at[mbuf], add=True)
        return carry
    lax.fori_loop(jnp.int32(0), jnp.int32(CHUNKS_PER_W), _step, jnp.int32(0))

    plsc.subcore_barrier()
    pltpu.sync_copy(cnt_sh.at[pl.ds(s * STRIPE, STRIPE)],
                    out_hbm.at[c].at[pl.ds(s * STRIPE, STRIPE)])


_deg_kernel = functools.partial(
    pl.kernel,
    out_type=jax.ShapeDtypeStruct((NC, N_PAD), jnp.float32),
    mesh=plsc.VectorSubcoreMesh(core_axis_name="c", subcore_axis_name="s"),
    scratch_types=[
        pltpu.VMEM((CHUNKS_PER_W * CH,), jnp.int32),
        pltpu.VMEM((CH,), jnp.int32),
        pltpu.VMEM((CH,), jnp.int32),
        pltpu.VMEM((CH,), jnp.float32),
        pltpu.VMEM((STRIPE,), jnp.float32),
        pltpu.VMEM_SHARED((N_PAD,), jnp.float32),
    ],
    compiler_params=pltpu.CompilerParams(use_tc_tiling_on_sc=False),
)(_deg_body)


DH = D_FEAT // 2
CHUNKS_B = E_PAD // (NS * CH)
NROW_T = N_NODES // NS


def _rsqrt16(v):
    i = plsc.bitcast(v, jnp.int32)
    i = jnp.full((16,), 0x5F3759DF, jnp.int32) - lax.shift_right_logical(
        i, jnp.full((16,), 1, jnp.int32))
    y = plsc.bitcast(i, jnp.float32)
    for _ in range(3):
        y = y * (1.5 - 0.5 * v * y * y)
    return y


def _prop_body(pk_hbm, x_hbm, cnt_hbm, out_hbm,
               pbuf, gbufa, gbufb, mbufa, mbufb, rowa, rowb, zbuf,
               dbuf, cbuf0, tmp,
               y_sh, acc_sh, sga, sgb, ssa, ssb):
    c = lax.axis_index("c")
    s = lax.axis_index("s")

    for r in range(16):
        for j in range(DH // 16):
            zbuf[r, pl.ds(j * 16, 16)] = jnp.zeros((16,), jnp.float32)

    def _z(i, carry):
        pltpu.sync_copy(zbuf, acc_sh.at[pl.ds(s * STRIPE + i * 16, 16)])
        return carry
    lax.fori_loop(jnp.int32(0), jnp.int32(STRIPE // 16), _z, jnp.int32(0))

    base = pl.multiple_of(lax.shift_left(
        lax.shift_right_logical(s * NROW_T, jnp.int32(3)), jnp.int32(3)), 8)
    shift = s * NROW_T - base
    pltpu.sync_copy(cnt_hbm.at[jnp.int32(0)].at[pl.ds(base, STRIPE)], cbuf0)
    pltpu.sync_copy(cnt_hbm.at[jnp.int32(1)].at[pl.ds(base, STRIPE)], dbuf)
    for g in range(STRIPE // 16):
        deg = cbuf0[pl.ds(g * 16, 16)] + dbuf[pl.ds(g * 16, 16)] + 1.0
        cbuf0[pl.ds(g * 16, 16)] = _rsqrt16(deg)

    pltpu.sync_copy(x_hbm.at[pl.ds(s * NROW_T, NROW_T), pl.ds(c * DH, DH)],
                    y_sh.at[pl.ds(s * NROW_T, NROW_T)])

    bc_dn = lax.GatherDimensionNumbers(
        offset_dims=(), collapsed_slice_dims=(0,), start_index_map=(0,))

    def _lane_bcast(v, i):
        return lax.gather(v, jnp.full((16, 1), i, jnp.int32), bc_dn, (1,),
                          mode=lax.GatherScatterMode.PROMISE_IN_BOUNDS)

    def _stage(b, carry):
        r0 = s * NROW_T + b * 125
        pltpu.sync_copy(y_sh.at[pl.ds(r0, 125)], tmp.at[pl.ds(0, 125)])
        for g in range(8):
            dvec = plsc.load_gather(
                cbuf0, [shift + b * 125 + g * 16 + lax.iota(jnp.int32, 16)])
            for i in range(16):
                d = _lane_bcast(dvec, i)
                for sl in range(4):
                    tmp[g * 16 + i, pl.ds(sl * 16, 16)] = (
                        d * tmp[g * 16 + i, pl.ds(sl * 16, 16)])
        pltpu.sync_copy(tmp.at[pl.ds(0, 125)], y_sh.at[pl.ds(r0, 125)])
        return carry
    lax.fori_loop(jnp.int32(0), jnp.int32(5), _stage, jnp.int32(0))

    nloc = CHUNKS_B * CH
    pltpu.sync_copy(pk_hbm.at[pl.ds(s * nloc, nloc)], pbuf)

    plsc.subcore_barrier()

    def _gather_start(gbuf, buf, sem):
        pltpu.async_copy(y_sh.at[gbuf], buf, sem)

    def _gather_wait(gbuf, buf, sem):
        pltpu.make_async_copy(y_sh.at[gbuf], buf, sem).wait()

    def _scatter_start(buf, mbuf, sem):
        pltpu.async_copy(buf, acc_sh.at[mbuf], sem, add=True)

    def _scatter_wait(buf, mbuf, sem):
        pltpu.make_async_copy(buf, acc_sh.at[mbuf], sem).wait()

    _unpack(pbuf, jnp.int32(0), gbufa, mbufa)
    _gather_start(gbufa, rowa, sga)
    _unpack(pbuf, jnp.int32(CH), gbufb, mbufb)
    _gather_start(gbufb, rowb, sgb)

    def _step(j, carry):
        k0 = 2 * j
        _gather_wait(gbufa, rowa, sga)
        _scatter_start(rowa, mbufa, ssa)

        _gather_wait(gbufb, rowb, sgb)
        _scatter_start(rowb, mbufb, ssb)

        @pl.when(j < CHUNKS_B // 2 - 1)
        def _():
            _scatter_wait(rowa, mbufa, ssa)
            _unpack(pbuf, (k0 + 2) * CH, gbufa, mbufa)
            _gather_start(gbufa, rowa, sga)

            _scatter_wait(rowb, mbufb, ssb)
            _unpack(pbuf, (k0 + 3) * CH, gbufb, mbufb)
            _gather_start(gbufb, rowb, sgb)
        return carry
    lax.fori_loop(jnp.int32(0), jnp.int32(CHUNKS_B // 2), _step, jnp.int32(0))

    _scatter_wait(rowa, mbufa, ssa)
    _scatter_wait(rowb, mbufb, ssb)

    plsc.subcore_barrier()

    def _out(i, carry):
        pltpu.sync_copy(acc_sh.at[pl.ds(s * STRIPE + i * CH, CH)], rowa)
        pltpu.sync_copy(rowa, out_hbm.at[c].at[pl.ds(s * STRIPE + i * CH, CH)])
        return carry
    lax.fori_loop(jnp.int32(0), jnp.int32(STRIPE // CH), _out, jnp.int32(0))


_prop_kernel = functools.partial(
    pl.kernel,
    out_type=jax.ShapeDtypeStruct((NC, N_PAD, DH), jnp.float32),
    mesh=plsc.VectorSubcoreMesh(core_axis_name="c", subcore_axis_name="s"),
    scratch_types=[
        pltpu.VMEM((CHUNKS_B * CH,), jnp.int32),
        pltpu.VMEM((CH,), jnp.int32),
        pltpu.VMEM((CH,), jnp.int32),
        pltpu.VMEM((CH,), jnp.int32),
        pltpu.VMEM((CH,), jnp.int32),
        pltpu.VMEM((CH, DH), jnp.float32),
        pltpu.VMEM((CH, DH), jnp.float32),
        pltpu.VMEM((16, DH), jnp.float32),
        pltpu.VMEM((STRIPE,), jnp.float32),
        pltpu.VMEM((STRIPE,), jnp.float32),
        pltpu.VMEM((CH, DH), jnp.float32),
        pltpu.VMEM_SHARED((N_NODES, DH), jnp.float32),
        pltpu.VMEM_SHARED((N_PAD, DH), jnp.float32),
        pltpu.SemaphoreType.DMA,
        pltpu.SemaphoreType.DMA,
        pltpu.SemaphoreType.DMA,
        pltpu.SemaphoreType.DMA,
    ],
    compiler_params=pltpu.CompilerParams(use_tc_tiling_on_sc=False,
                                         needs_layout_passes=False),
)(_prop_body)


def _e3_body(p_ref, x_ref, cnt_ref, out_ref):
    dinv = lax.rsqrt(cnt_ref[0] + cnt_ref[1] + 1.0)
    x = x_ref[...]
    out_ref[:, 0:64] = dinv * p_ref[0] + dinv * dinv * x[:, 0:64]
    out_ref[:, 64:128] = dinv * p_ref[1] + dinv * dinv * x[:, 64:128]


def kernel(x, edge_index):
    ei = edge_index.astype(jnp.int32)
    row, col = ei[0], ei[1]
    pad = E_PAD - N_EDGES
    pad_col = DUMMY + (jnp.arange(pad, dtype=jnp.int32) % (N_PAD - N_NODES))
    row = jnp.concatenate([row, jnp.zeros((pad,), jnp.int32)])
    col = jnp.concatenate([col, pad_col])
    packed = jnp.bitwise_or(jnp.left_shift(col, 16), row)

    cnt = _deg_kernel(packed)

    p = _prop_kernel(packed, x, cnt)

    out = pl.pallas_call(
        _e3_body,
        out_shape=jax.ShapeDtypeStruct((N_NODES, D_FEAT), jnp.float32),
    )(p[:, :N_NODES, :], x, cnt[:, :N_NODES, None])
    return out


# speedup vs baseline: 2.7170x; 1.1090x over previous
import functools

import jax
import jax.numpy as jnp
from jax import lax
from jax.experimental import pallas as pl
from jax.experimental.pallas import tpu as pltpu
from jax.experimental.pallas import tpu_sc as plsc

N_NODES = 10000
N_EDGES = 320000
D_FEAT = 128

NC = 2
NS = 16
NW = NC * NS
CH = 128
CHUNKS_PER_W = 80
E_PAD = NW * CHUNKS_PER_W * CH
N_PAD = 10240
STRIPE = N_PAD // NS
DUMMY = N_NODES


def _unpack(pbuf, base, gbuf, mbuf):
    for i in range(CH // 16):
        v = pbuf[pl.ds(base + i * 16, 16)]
        r = v & jnp.int32(0xFFFF)
        cc = lax.shift_right_logical(v, jnp.int32(16))
        gbuf[pl.ds(i * 16, 16)] = r
        mbuf[pl.ds(i * 16, 16)] = jnp.where(
            r == cc, jnp.full((16,), DUMMY, jnp.int32), cc)


def _deg_body(pk_hbm, out_hbm, pbuf, gbuf, mbuf, vbuf, zbuf, cnt_sh):
    c = lax.axis_index("c")
    s = lax.axis_index("s")
    w = s * NC + c

    def _z(i, carry):
        zbuf[pl.ds(i * 16, 16)] = jnp.zeros((16,), jnp.float32)
        return carry
    lax.fori_loop(jnp.int32(0), jnp.int32(STRIPE // 16), _z, jnp.int32(0))
    pltpu.sync_copy(zbuf, cnt_sh.at[pl.ds(s * STRIPE, STRIPE)])

    for i in range(CH // 16):
        vbuf[pl.ds(i * 16, 16)] = jnp.ones((16,), jnp.float32)

    nloc = CHUNKS_PER_W * CH
    pltpu.sync_copy(pk_hbm.at[pl.ds(w * nloc, nloc)], pbuf)

    plsc.subcore_barrier()

    def _step(k, carry):
        _unpack(pbuf, k * CH, gbuf, mbuf)
        pltpu.sync_copy(vbuf, cnt_sh.at[mbuf], add=True)
        return carry
    lax.fori_loop(jnp.int32(0), jnp.int32(CHUNKS_PER_W), _step, jnp.int32(0))

    plsc.subcore_barrier()
    pltpu.sync_copy(cnt_sh.at[pl.ds(s * STRIPE, STRIPE)],
                    out_hbm.at[c].at[pl.ds(s * STRIPE, STRIPE)])


_deg_kernel = functools.partial(
    pl.kernel,
    out_type=jax.ShapeDtypeStruct((NC, N_PAD), jnp.float32),
    mesh=plsc.VectorSubcoreMesh(core_axis_name="c", subcore_axis_name="s"),
    scratch_types=[
        pltpu.VMEM((CHUNKS_PER_W * CH,), jnp.int32),
        pltpu.VMEM((CH,), jnp.int32),
        pltpu.VMEM((CH,), jnp.int32),
        pltpu.VMEM((CH,), jnp.float32),
        pltpu.VMEM((STRIPE,), jnp.float32),
        pltpu.VMEM_SHARED((N_PAD,), jnp.float32),
    ],
    compiler_params=pltpu.CompilerParams(use_tc_tiling_on_sc=False),
)(_deg_body)


DH = D_FEAT // 2
CHUNKS_B = E_PAD // (NS * CH)
NROW_T = N_NODES // NS


def _rsqrt16(v):
    i = plsc.bitcast(v, jnp.int32)
    i = jnp.full((16,), 0x5F3759DF, jnp.int32) - lax.shift_right_logical(
        i, jnp.full((16,), 1, jnp.int32))
    y = plsc.bitcast(i, jnp.float32)
    for _ in range(3):
        y = y * (1.5 - 0.5 * v * y * y)
    return y


def _prop_body(pk_hbm, x_hbm, cnt_hbm, out_hbm,
               pbuf, gbufa, gbufb, mbufa, mbufb, rowa, rowb, zbuf,
               dbuf, cbuf0, tmp,
               y_sh, acc_sh, sga, sgb, ssa, ssb):
    c = lax.axis_index("c")
    s = lax.axis_index("s")

    for r in range(16):
        for j in range(DH // 16):
            zbuf[r, pl.ds(j * 16, 16)] = jnp.zeros((16,), jnp.float32)

    def _z(i, carry):
        pltpu.sync_copy(zbuf, acc_sh.at[pl.ds(s * STRIPE + i * 16, 16)])
        return carry
    lax.fori_loop(jnp.int32(0), jnp.int32(STRIPE // 16), _z, jnp.int32(0))

    base = pl.multiple_of(lax.shift_left(
        lax.shift_right_logical(s * NROW_T, jnp.int32(3)), jnp.int32(3)), 8)
    shift = s * NROW_T - base
    pltpu.sync_copy(cnt_hbm.at[jnp.int32(0)].at[pl.ds(base, STRIPE)], cbuf0)
    pltpu.sync_copy(cnt_hbm.at[jnp.int32(1)].at[pl.ds(base, STRIPE)], dbuf)
    for g in range(STRIPE // 16):
        deg = cbuf0[pl.ds(g * 16, 16)] + dbuf[pl.ds(g * 16, 16)] + 1.0
        cbuf0[pl.ds(g * 16, 16)] = _rsqrt16(deg)

    pltpu.sync_copy(x_hbm.at[pl.ds(s * NROW_T, NROW_T), pl.ds(c * DH, DH)],
                    y_sh.at[pl.ds(s * NROW_T, NROW_T)])

    bc_dn = lax.GatherDimensionNumbers(
        offset_dims=(), collapsed_slice_dims=(0,), start_index_map=(0,))

    def _lane_bcast(v, i):
        return lax.gather(v, jnp.full((16, 1), i, jnp.int32), bc_dn, (1,),
                          mode=lax.GatherScatterMode.PROMISE_IN_BOUNDS)

    def _stage(b, carry):
        r0 = s * NROW_T + b * 125
        pltpu.sync_copy(y_sh.at[pl.ds(r0, 125)], tmp.at[pl.ds(0, 125)])
        for g in range(8):
            dvec = plsc.load_gather(
                cbuf0, [shift + b * 125 + g * 16 + lax.iota(jnp.int32, 16)])
            for i in range(16):
                d = _lane_bcast(dvec, i)
                for sl in range(4):
                    tmp[g * 16 + i, pl.ds(sl * 16, 16)] = (
                        d * tmp[g * 16 + i, pl.ds(sl * 16, 16)])
        pltpu.sync_copy(tmp.at[pl.ds(0, 125)], y_sh.at[pl.ds(r0, 125)])
        return carry
    lax.fori_loop(jnp.int32(0), jnp.int32(5), _stage, jnp.int32(0))

    nloc = CHUNKS_B * CH
    pltpu.sync_copy(pk_hbm.at[pl.ds(s * nloc, nloc)], pbuf)

    plsc.subcore_barrier()

    def _gather_start(gbuf, buf, sem):
        pltpu.async_copy(y_sh.at[gbuf], buf, sem)

    def _gather_wait(gbuf, buf, sem):
        pltpu.make_async_copy(y_sh.at[gbuf], buf, sem).wait()

    def _scatter_start(buf, mbuf, sem):
        pltpu.async_copy(buf, acc_sh.at[mbuf], sem, add=True)

    def _scatter_wait(buf, mbuf, sem):
        pltpu.make_async_copy(buf, acc_sh.at[mbuf], sem).wait()

    _unpack(pbuf, jnp.int32(0), gbufa, mbufa)
    _gather_start(gbufa, rowa, sga)
    _unpack(pbuf, jnp.int32(CH), gbufb, mbufb)
    _gather_start(gbufb, rowb, sgb)

    def _step(j, carry):
        k0 = 2 * j
        _gather_wait(gbufa, rowa, sga)
        _scatter_start(rowa, mbufa, ssa)

        _gather_wait(gbufb, rowb, sgb)
        _scatter_start(rowb, mbufb, ssb)

        @pl.when(j < CHUNKS_B // 2 - 1)
        def _():
            _scatter_wait(rowa, mbufa, ssa)
            _unpack(pbuf, (k0 + 2) * CH, gbufa, mbufa)
            _gather_start(gbufa, rowa, sga)

            _scatter_wait(rowb, mbufb, ssb)
            _unpack(pbuf, (k0 + 3) * CH, gbufb, mbufb)
            _gather_start(gbufb, rowb, sgb)
        return carry
    lax.fori_loop(jnp.int32(0), jnp.int32(CHUNKS_B // 2), _step, jnp.int32(0))

    _scatter_wait(rowa, mbufa, ssa)
    _scatter_wait(rowb, mbufb, ssb)

    plsc.subcore_barrier()

    def _out(b, carry):
        r0 = s * NROW_T + b * 125
        pltpu.sync_copy(acc_sh.at[pl.ds(r0, 125)], tmp.at[pl.ds(0, 125)])
        pltpu.sync_copy(y_sh.at[pl.ds(r0, 125)], rowa.at[pl.ds(0, 125)])
        for g in range(8):
            dvec = plsc.load_gather(
                cbuf0, [shift + b * 125 + g * 16 + lax.iota(jnp.int32, 16)])
            for i in range(16):
                d = _lane_bcast(dvec, i)
                for sl in range(4):
                    tmp[g * 16 + i, pl.ds(sl * 16, 16)] = d * (
                        tmp[g * 16 + i, pl.ds(sl * 16, 16)]
                        + rowa[g * 16 + i, pl.ds(sl * 16, 16)])
        pltpu.sync_copy(tmp.at[pl.ds(0, 125)],
                        out_hbm.at[pl.ds(r0, 125), pl.ds(c * DH, DH)])
        return carry
    lax.fori_loop(jnp.int32(0), jnp.int32(5), _out, jnp.int32(0))


_prop_kernel = functools.partial(
    pl.kernel,
    out_type=jax.ShapeDtypeStruct((N_NODES, D_FEAT), jnp.float32),
    mesh=plsc.VectorSubcoreMesh(core_axis_name="c", subcore_axis_name="s"),
    scratch_types=[
        pltpu.VMEM((CHUNKS_B * CH,), jnp.int32),
        pltpu.VMEM((CH,), jnp.int32),
        pltpu.VMEM((CH,), jnp.int32),
        pltpu.VMEM((CH,), jnp.int32),
        pltpu.VMEM((CH,), jnp.int32),
        pltpu.VMEM((CH, DH), jnp.float32),
        pltpu.VMEM((CH, DH), jnp.float32),
        pltpu.VMEM((16, DH), jnp.float32),
        pltpu.VMEM((STRIPE,), jnp.float32),
        pltpu.VMEM((STRIPE,), jnp.float32),
        pltpu.VMEM((CH, DH), jnp.float32),
        pltpu.VMEM_SHARED((N_NODES, DH), jnp.float32),
        pltpu.VMEM_SHARED((N_PAD, DH), jnp.float32),
        pltpu.SemaphoreType.DMA,
        pltpu.SemaphoreType.DMA,
        pltpu.SemaphoreType.DMA,
        pltpu.SemaphoreType.DMA,
    ],
    compiler_params=pltpu.CompilerParams(use_tc_tiling_on_sc=False,
                                         needs_layout_passes=False),
)(_prop_body)


def _pack_body(r_ref, c_ref, o_ref):
    o_ref[...] = jnp.bitwise_or(
        jnp.left_shift(c_ref[...], 16), r_ref[...])


def kernel(x, edge_index):
    ei = edge_index.astype(jnp.int32)
    row, col = ei[0], ei[1]
    pad = E_PAD - N_EDGES
    pad_col = DUMMY + (jnp.arange(pad, dtype=jnp.int32) % (N_PAD - N_NODES))
    row = jnp.concatenate([row, jnp.zeros((pad,), jnp.int32)])
    col = jnp.concatenate([col, pad_col])

    packed = pl.pallas_call(
        _pack_body,
        out_shape=jax.ShapeDtypeStruct((E_PAD // 128, 128), jnp.int32),
    )(row.reshape(E_PAD // 128, 128), col.reshape(E_PAD // 128, 128))
    packed = packed.reshape(E_PAD)

    cnt = _deg_kernel(packed)
    return _prop_kernel(packed, x, cnt)


# speedup vs baseline: 3.0972x; 1.1399x over previous
import functools

import jax
import jax.numpy as jnp
from jax import lax
from jax.experimental import pallas as pl
from jax.experimental.pallas import tpu as pltpu
from jax.experimental.pallas import tpu_sc as plsc

N_NODES = 10000
N_EDGES = 320000
D_FEAT = 128

NC = 2
NS = 16
NW = NC * NS
CH = 128
CHUNKS_PER_W = 80
E_PAD = NW * CHUNKS_PER_W * CH
N_PAD = 10240
STRIPE = N_PAD // NS
DUMMY = N_NODES


def _unpack(pbuf, base, gbuf, mbuf):
    for i in range(CH // 16):
        v = pbuf[pl.ds(base + i * 16, 16)]
        r = v & jnp.int32(0xFFFF)
        cc = lax.shift_right_logical(v, jnp.int32(16))
        gbuf[pl.ds(i * 16, 16)] = r
        mbuf[pl.ds(i * 16, 16)] = jnp.where(
            r == cc, jnp.full((16,), DUMMY, jnp.int32), cc)


def _deg_body(pk_hbm, out_hbm, pbuf, gbuf, mbuf, vbuf, zbuf, cnt_sh):
    c = lax.axis_index("c")
    s = lax.axis_index("s")
    w = s * NC + c

    def _z(i, carry):
        zbuf[pl.ds(i * 16, 16)] = jnp.zeros((16,), jnp.float32)
        return carry
    lax.fori_loop(jnp.int32(0), jnp.int32(STRIPE // 16), _z, jnp.int32(0))
    pltpu.sync_copy(zbuf, cnt_sh.at[pl.ds(s * STRIPE, STRIPE)])

    for i in range(CH // 16):
        vbuf[pl.ds(i * 16, 16)] = jnp.ones((16,), jnp.float32)

    nloc = CHUNKS_PER_W * CH
    pltpu.sync_copy(pk_hbm.at[pl.ds(w * nloc, nloc)], pbuf)

    plsc.subcore_barrier()

    def _step(k, carry):
        _unpack(pbuf, k * CH, gbuf, mbuf)
        pltpu.sync_copy(vbuf, cnt_sh.at[mbuf], add=True)
        return carry
    lax.fori_loop(jnp.int32(0), jnp.int32(CHUNKS_PER_W), _step, jnp.int32(0))

    plsc.subcore_barrier()
    pltpu.sync_copy(cnt_sh.at[pl.ds(s * STRIPE, STRIPE)],
                    out_hbm.at[c].at[pl.ds(s * STRIPE, STRIPE)])


_deg_kernel = functools.partial(
    pl.kernel,
    out_type=jax.ShapeDtypeStruct((NC, N_PAD), jnp.float32),
    mesh=plsc.VectorSubcoreMesh(core_axis_name="c", subcore_axis_name="s"),
    scratch_types=[
        pltpu.VMEM((CHUNKS_PER_W * CH,), jnp.int32),
        pltpu.VMEM((CH,), jnp.int32),
        pltpu.VMEM((CH,), jnp.int32),
        pltpu.VMEM((CH,), jnp.float32),
        pltpu.VMEM((STRIPE,), jnp.float32),
        pltpu.VMEM_SHARED((N_PAD,), jnp.float32),
    ],
    compiler_params=pltpu.CompilerParams(use_tc_tiling_on_sc=False),
)(_deg_body)


DH = D_FEAT // 2
CHUNKS_B = E_PAD // (NS * CH)
NROW_T = N_NODES // NS


def _rsqrt16(v):
    i = plsc.bitcast(v, jnp.int32)
    i = jnp.full((16,), 0x5F3759DF, jnp.int32) - lax.shift_right_logical(
        i, jnp.full((16,), 1, jnp.int32))
    y = plsc.bitcast(i, jnp.float32)
    for _ in range(3):
        y = y * (1.5 - 0.5 * v * y * y)
    return y


def _prop_body(pk_hbm, x_hbm, out_hbm,
               pbuf, gbufa, gbufb, mbufa, mbufb,
               rowa, rowb, zbuf, dbuf, cbuf0, tmp,
               cnt_sh, y_sh, acc_sh, sga, sgb, ssa, ssb):
    c = lax.axis_index("c")
    s = lax.axis_index("s")

    for r in range(16):
        for j in range(DH // 16):
            zbuf[r, pl.ds(j * 16, 16)] = jnp.zeros((16,), jnp.float32)

    def _z(i, carry):
        pltpu.sync_copy(zbuf, acc_sh.at[pl.ds(s * STRIPE + i * 16, 16)])
        return carry
    lax.fori_loop(jnp.int32(0), jnp.int32(STRIPE // 16), _z, jnp.int32(0))

    base = pl.multiple_of(lax.shift_left(
        lax.shift_right_logical(s * NROW_T, jnp.int32(3)), jnp.int32(3)), 8)
    shift = s * NROW_T - base

    def _zc(i, carry):
        cbuf0[pl.ds(i * 16, 16)] = jnp.zeros((16,), jnp.float32)
        dbuf[pl.ds(i * 16, 16)] = jnp.ones((16,), jnp.float32)
        return carry
    lax.fori_loop(jnp.int32(0), jnp.int32(STRIPE // 16), _zc, jnp.int32(0))
    pltpu.sync_copy(cbuf0, cnt_sh.at[pl.ds(s * STRIPE, STRIPE)])

    pltpu.sync_copy(x_hbm.at[pl.ds(s * NROW_T, NROW_T), pl.ds(c * DH, DH)],
                    y_sh.at[pl.ds(s * NROW_T, NROW_T)])

    bc_dn = lax.GatherDimensionNumbers(
        offset_dims=(), collapsed_slice_dims=(0,), start_index_map=(0,))

    def _lane_bcast(v, i):
        return lax.gather(v, jnp.full((16, 1), i, jnp.int32), bc_dn, (1,),
                          mode=lax.GatherScatterMode.PROMISE_IN_BOUNDS)


    nloc = CHUNKS_B * CH
    pltpu.sync_copy(pk_hbm.at[pl.ds(s * nloc, nloc)], pbuf)

    plsc.subcore_barrier()

    def _gather_start(gbuf, buf, sem):
        pltpu.async_copy(y_sh.at[gbuf], buf, sem)

    def _gather_wait(gbuf, buf, sem):
        pltpu.make_async_copy(y_sh.at[gbuf], buf, sem).wait()

    def _scatter_start(buf, mbuf, sem):
        pltpu.async_copy(buf, acc_sh.at[mbuf], sem, add=True)

    def _scatter_wait(buf, mbuf, sem):
        pltpu.make_async_copy(buf, acc_sh.at[mbuf], sem).wait()

    slots = ((gbufa, mbufa, rowa, sga, ssa), (gbufb, mbufb, rowb, sgb, ssb))
    for q, (gb, mb, rw, sg, ss) in enumerate(slots):
        _unpack(pbuf, jnp.int32(q * CH), gb, mb)
        _gather_start(gb, rw, sg)

    def _step(j, carry):
        k0 = 2 * j
        for q, (gb, mb, rw, sg, ss) in enumerate(slots):
            _gather_wait(gb, rw, sg)
            _scatter_start(rw, mb, ss)

        @pl.when(j < CHUNKS_B // 2 - 1)
        def _():
            for q, (gb, mb, rw, sg, ss) in enumerate(slots):
                _scatter_wait(rw, mb, ss)
                _unpack(pbuf, (k0 + 2 + q) * CH, gb, mb)
                _gather_start(gb, rw, sg)
        return carry
    lax.fori_loop(jnp.int32(0), jnp.int32(CHUNKS_B // 2), _step, jnp.int32(0))

    for q, (gb, mb, rw, sg, ss) in enumerate(slots):
        _scatter_wait(rw, mb, ss)

    plsc.subcore_barrier()

    def _out(b, carry):
        r0 = s * NROW_T + b * 125
        pltpu.sync_copy(acc_sh.at[pl.ds(r0, 125)], tmp.at[pl.ds(0, 125)])
        pltpu.sync_copy(y_sh.at[pl.ds(r0, 125)], rowa.at[pl.ds(0, 125)])
        for g in range(8):
            dvec = plsc.load_gather(
                cbuf0, [shift + b * 125 + g * 16 + lax.iota(jnp.int32, 16)])
            for i in range(16):
                d = _lane_bcast(dvec, i)
                for sl in range(4):
                    tmp[g * 16 + i, pl.ds(sl * 16, 16)] = d * (
                        tmp[g * 16 + i, pl.ds(sl * 16, 16)]
                        + rowa[g * 16 + i, pl.ds(sl * 16, 16)])
        pltpu.sync_copy(tmp.at[pl.ds(0, 125)],
                        out_hbm.at[pl.ds(r0, 125), pl.ds(c * DH, DH)])
        return carry
    lax.fori_loop(jnp.int32(0), jnp.int32(5), _out, jnp.int32(0))


_prop_kernel = functools.partial(
    pl.kernel,
    out_type=jax.ShapeDtypeStruct((N_NODES, D_FEAT), jnp.float32),
    mesh=plsc.VectorSubcoreMesh(core_axis_name="c", subcore_axis_name="s"),
    scratch_types=[
        pltpu.VMEM((CHUNKS_B * CH,), jnp.int32),
        pltpu.VMEM((CH,), jnp.int32),
        pltpu.VMEM((CH,), jnp.int32),
        pltpu.VMEM((CH,), jnp.int32),
        pltpu.VMEM((CH,), jnp.int32),
        pltpu.VMEM((CH, DH), jnp.float32),
        pltpu.VMEM((CH, DH), jnp.float32),
        pltpu.VMEM((16, DH), jnp.float32),
        pltpu.VMEM((STRIPE,), jnp.float32),
        pltpu.VMEM((STRIPE,), jnp.float32),
        pltpu.VMEM((CH, DH), jnp.float32),
        pltpu.VMEM_SHARED((N_PAD,), jnp.float32),
        pltpu.VMEM_SHARED((N_NODES, DH), jnp.float32),
        pltpu.VMEM_SHARED((N_PAD, DH), jnp.float32),
        pltpu.SemaphoreType.DMA,
        pltpu.SemaphoreType.DMA,
        pltpu.SemaphoreType.DMA,
        pltpu.SemaphoreType.DMA,
    ],
    compiler_params=pltpu.CompilerParams(use_tc_tiling_on_sc=False,
                                         needs_layout_passes=False),
)(_prop_body)


def _pack_body(r_ref, c_ref, o_ref):
    o_ref[...] = jnp.bitwise_or(
        jnp.left_shift(c_ref[...], 16), r_ref[...])


def kernel(x, edge_index):
    ei = edge_index.astype(jnp.int32)
    row, col = ei[0], ei[1]
    pad = E_PAD - N_EDGES
    pad_col = DUMMY + (jnp.arange(pad, dtype=jnp.int32) % (N_PAD - N_NODES))
    row = jnp.concatenate([row, jnp.zeros((pad,), jnp.int32)])
    col = jnp.concatenate([col, pad_col])

    packed = pl.pallas_call(
        _pack_body,
        out_shape=jax.ShapeDtypeStruct((E_PAD // 128, 128), jnp.int32),
    )(row.reshape(E_PAD // 128, 128), col.reshape(E_PAD // 128, 128))
    packed = packed.reshape(E_PAD)

    return _prop_kernel(packed, x)
